# baseline (speedup 1.0000x reference)
import jax
import jax.numpy as jnp
from jax.experimental import pallas as pl

N = 10000
E = 160000
IN = 128
H = 256
HEADS = 4
NC = 10
NE = 4
NS = 3


def _copy_kernel(x_ref, o_ref):
    o_ref[...] = x_ref[...]


def kernel(x, edge_index, edge_type, ggc_W, ggc_b, gru_Wih, gru_Whh, gru_bih, gru_bhh,
           gat_Wsrc, gat_bsrc, gat_Wdst, gat_bdst, gat_attn, gat_bias,
           skip_W, skip_b, bn1_g, bn1_b, bn2_g, bn2_b,
           red_W, red_b, gate_W, gate_b, cls_W1, cls_b1, cls_W2, cls_b2):
    src = edge_index[0]
    dst = edge_index[1]
    Nn = x.shape[0]
    h = jnp.pad(x, ((0, 0), (0, H - x.shape[1])))
    for _ in range(NS):
        proj = jnp.einsum('ni,eoi->neo', h, ggc_W) + ggc_b[None, :, :]
        msgs = proj[src, edge_type]
        a = jax.ops.segment_sum(msgs, dst, num_segments=N)
        gi = a @ gru_Wih.T + gru_bih
        gh = h @ gru_Whh.T + gru_bhh
        i_r, i_z, i_n = jnp.split(gi, 3, axis=1)
        h_r, h_z, h_n = jnp.split(gh, 3, axis=1)
        r = jax.nn.sigmoid(i_r + h_r)
        z = jax.nn.sigmoid(i_z + h_z)
        n = jnp.tanh(i_n + r * h_n)
        h = (1.0 - z) * n + z * h
    h2 = jax.nn.elu(h)
    h2 = (h2 - h2.mean(0)) / jnp.sqrt(h2.var(0) + 1e-5) * bn2_g + bn2_b
    fsrc = (h2 @ gat_Wsrc.T + gat_bsrc).reshape(Nn, HEADS, H)
    fdst = (h2 @ gat_Wdst.T + gat_bdst).reshape(Nn, HEADS, H)
    e = jax.nn.leaky_relu(fsrc[src] + fdst[dst], negative_slope=0.2)
    logits = (e * gat_attn[None, :, :]).sum(-1)
    m = jax.ops.segment_max(logits, dst, num_segments=N)
    m = jnp.where(jnp.isfinite(m), m, 0.0)
    ex = jnp.exp(logits - m[dst])
    denom = jax.ops.segment_sum(ex, dst, num_segments=N)
    alpha = ex / jnp.maximum(denom[dst], 1e-9)
    out = jax.ops.segment_sum(alpha[:, :, None] * fsrc[src], dst, num_segments=N)
    out = out + gat_bias.reshape(1, HEADS, H)
    h1 = jax.nn.elu(out).reshape(Nn, HEADS * H)
    skip = x @ skip_W.T + skip_b
    h1 = h1 + skip
    h1 = (h1 - h1.mean(0)) / jnp.sqrt(h1.var(0) + 1e-5) * bn1_g + bn1_b
    hr = h1 @ red_W.T + red_b
    gate = hr @ gate_W.T + gate_b
    gate = jax.nn.softmax(gate, axis=0)
    h_g = (gate * hr).sum(0, keepdims=True)
    z1 = jax.nn.relu(h_g @ cls_W1.T + cls_b1)
    out = z1 @ cls_W2.T + cls_b2
    return pl.pallas_call(
        _copy_kernel,
        out_shape=jax.ShapeDtypeStruct(out.shape, out.dtype),
    )(out)


# speedup vs baseline: 1.2618x; 1.2618x over previous
import functools

import jax
import jax.numpy as jnp
from jax import lax
from jax.experimental import pallas as pl
from jax.experimental.pallas import tpu as pltpu
from jax.experimental.pallas import tpu_sc as plsc

N = 10000
E = 160000
IN = 128
H = 256
HEADS = 4
NCLS = 10
NE = 4
NS = 3

NP = 10240
CH = 80
TILES = 16
CORES = 2
EPT = E // TILES
NPT = NP // TILES


def _ggc_edge_sc(table, eidx2, dst, zeros):
    mesh = plsc.VectorSubcoreMesh(
        core_axis_name="c", subcore_axis_name="s",
        num_cores=CORES, num_subcores=TILES)

    @functools.partial(
        pl.kernel,
        out_type=jax.ShapeDtypeStruct((CORES, NP, 128), jnp.float32),
        mesh=mesh,
        scratch_types=[
            pltpu.VMEM((CH,), jnp.int32),
            pltpu.VMEM((CH,), jnp.int32),
            pltpu.VMEM((CH, 128), jnp.float32),
            pltpu.VMEM_SHARED((NP, 128), jnp.float32),
            pltpu.SemaphoreType.DMA,
        ],
    )
    def k(table_h, eidx_h, dst_h, zeros_h, out_h, idx_v, dst_v, rows_v, acc, sem):
        c = lax.axis_index("c")
        s = lax.axis_index("s")
        pltpu.sync_copy(zeros_h, acc.at[pl.ds(s * NPT, NPT)])
        plsc.subcore_barrier()

        def chunk(i, carry):
            base = s * EPT + i * CH
            pltpu.sync_copy(eidx_h.at[pl.ds(c * E + base, CH)], idx_v)
            pltpu.sync_copy(dst_h.at[pl.ds(base, CH)], dst_v)
            pltpu.async_copy(table_h.at[idx_v], rows_v, sem).wait()
            pltpu.sync_copy(rows_v, acc.at[dst_v], add=True)
            return carry

        lax.fori_loop(0, EPT // CH, chunk, 0)
        plsc.subcore_barrier()
        pltpu.sync_copy(acc.at[pl.ds(s * NPT, NPT)],
                        out_h.at[c, pl.ds(s * NPT, NPT)])

    return k(table, eidx2, dst, zeros)


def kernel(x, edge_index, edge_type, ggc_W, ggc_b, gru_Wih, gru_Whh, gru_bih, gru_bhh,
           gat_Wsrc, gat_bsrc, gat_Wdst, gat_bdst, gat_attn, gat_bias,
           skip_W, skip_b, bn1_g, bn1_b, bn2_g, bn2_b,
           red_W, red_b, gate_W, gate_b, cls_W1, cls_b1, cls_W2, cls_b2):
    src = edge_index[0]
    dst = edge_index[1]
    zeros = jnp.zeros((NPT, 128), jnp.float32)
    eidx = edge_type.astype(jnp.int32) * NP + src
    eidx2 = jnp.concatenate([eidx, eidx + NE * NP])

    h = jnp.pad(x, ((0, NP - N), (0, H - IN)))
    for _ in range(NS):
        proj = jnp.einsum('ni,toi->tno', h, ggc_W) + ggc_b[:, None, :]
        table = proj.reshape(NE, NP, 2, 128).transpose(2, 0, 1, 3).reshape(-1, 128)
        a2 = _ggc_edge_sc(table, eidx2, dst, zeros)
        a = jnp.concatenate([a2[0, :N], a2[1, :N]], axis=1)
        hh = h[:N]
        gi = a @ gru_Wih.T + gru_bih
        gh = hh @ gru_Whh.T + gru_bhh
        i_r, i_z, i_n = jnp.split(gi, 3, axis=1)
        h_r, h_z, h_n = jnp.split(gh, 3, axis=1)
        r = jax.nn.sigmoid(i_r + h_r)
        z = jax.nn.sigmoid(i_z + h_z)
        n = jnp.tanh(i_n + r * h_n)
        hnew = (1.0 - z) * n + z * hh
        h = jnp.pad(hnew, ((0, NP - N), (0, 0)))
    h = h[:N]
    h2 = jax.nn.elu(h)
    h2 = (h2 - h2.mean(0)) / jnp.sqrt(h2.var(0) + 1e-5) * bn2_g + bn2_b
    fsrc = (h2 @ gat_Wsrc.T + gat_bsrc).reshape(N, HEADS, H)
    fdst = (h2 @ gat_Wdst.T + gat_bdst).reshape(N, HEADS, H)
    e = jax.nn.leaky_relu(fsrc[src] + fdst[dst], negative_slope=0.2)
    logits = (e * gat_attn[None, :, :]).sum(-1)
    m = jax.ops.segment_max(logits, dst, num_segments=N)
    m = jnp.where(jnp.isfinite(m), m, 0.0)
    ex = jnp.exp(logits - m[dst])
    denom = jax.ops.segment_sum(ex, dst, num_segments=N)
    alpha = ex / jnp.maximum(denom[dst], 1e-9)
    out = jax.ops.segment_sum(alpha[:, :, None] * fsrc[src], dst, num_segments=N)
    out = out + gat_bias.reshape(1, HEADS, H)
    h1 = jax.nn.elu(out).reshape(N, HEADS * H)
    skip = x @ skip_W.T + skip_b
    h1 = h1 + skip
    h1 = (h1 - h1.mean(0)) / jnp.sqrt(h1.var(0) + 1e-5) * bn1_g + bn1_b
    hr = h1 @ red_W.T + red_b
    gate = hr @ gate_W.T + gate_b
    gate = jax.nn.softmax(gate, axis=0)
    h_g = (gate * hr).sum(0, keepdims=True)
    z1 = jax.nn.relu(h_g @ cls_W1.T + cls_b1)
    return z1 @ cls_W2.T + cls_b2


# speedup vs baseline: 5.3458x; 4.2366x over previous
import functools

import jax
import jax.numpy as jnp
from jax import lax
from jax.experimental import pallas as pl
from jax.experimental.pallas import tpu as pltpu
from jax.experimental.pallas import tpu_sc as plsc

N = 10000
E = 160000
IN = 128
H = 256
HEADS = 4
NCLS = 10
NE = 4
NS = 3

NP = 10240
CH = 80
TILES = 16
CORES = 2
EPT = E // TILES
NPT = NP // TILES


def _ggc_edge_sc(table, eidx2, dst, zeros):
    mesh = plsc.VectorSubcoreMesh(
        core_axis_name="c", subcore_axis_name="s",
        num_cores=CORES, num_subcores=TILES)

    @functools.partial(
        pl.kernel,
        out_type=jax.ShapeDtypeStruct((CORES, NP, 128), jnp.float32),
        mesh=mesh,
        compiler_params=pltpu.CompilerParams(needs_layout_passes=False),
        scratch_types=[
            pltpu.VMEM((CH,), jnp.int32),
            pltpu.VMEM((CH,), jnp.int32),
            pltpu.VMEM((CH, 128), jnp.float32),
            pltpu.VMEM_SHARED((NP, 128), jnp.float32),
            pltpu.SemaphoreType.DMA,
        ],
    )
    def k(table_h, eidx_h, dst_h, zeros_h, out_h, idx_v, dst_v, rows_v, acc, sem):
        c = lax.axis_index("c")
        s = lax.axis_index("s")
        pltpu.sync_copy(zeros_h, acc.at[pl.ds(s * NPT, NPT)])
        plsc.subcore_barrier()

        def chunk(i, carry):
            base = s * EPT + i * CH
            pltpu.sync_copy(eidx_h.at[pl.ds(c * E + base, CH)], idx_v)
            pltpu.sync_copy(dst_h.at[pl.ds(base, CH)], dst_v)
            pltpu.async_copy(table_h.at[idx_v], rows_v, sem).wait()
            pltpu.sync_copy(rows_v, acc.at[dst_v], add=True)
            return carry

        lax.fori_loop(0, EPT // CH, chunk, 0)
        plsc.subcore_barrier()
        pltpu.sync_copy(acc.at[pl.ds(s * NPT, NPT)],
                        out_h.at[c, pl.ds(s * NPT, NPT)])

    return k(table, eidx2, dst, zeros)


def _gat_logits_sc(fsrc_t, fdst_t, fidx_src, fidx_dst, dst, attn_t, zeros_flat):
    mesh = plsc.VectorSubcoreMesh(
        core_axis_name="c", subcore_axis_name="s",
        num_cores=CORES, num_subcores=TILES)
    @functools.partial(
        pl.kernel,
        out_type=(jax.ShapeDtypeStruct((4 * E,), jnp.float32),
                  jax.ShapeDtypeStruct((4 * NP,), jnp.float32)),
        mesh=mesh,
        compiler_params=pltpu.CompilerParams(needs_layout_passes=False),
        scratch_types=[
            pltpu.VMEM((CH,), jnp.int32),
            pltpu.VMEM((CH,), jnp.int32),
            pltpu.VMEM((CH, 128), jnp.float32),
            pltpu.VMEM((CH, 128), jnp.float32),
            pltpu.VMEM((256,), jnp.float32),
            pltpu.VMEM((512,), jnp.float32),
            pltpu.VMEM((2 * EPT,), jnp.float32),
            pltpu.VMEM((2 * EPT,), jnp.float32),
            pltpu.VMEM((EPT,), jnp.int32),
            pltpu.VMEM((2 * NP,), jnp.float32),
            pltpu.VMEM((1280,), jnp.float32),
            pltpu.VMEM((1280,), jnp.float32),
            pltpu.VMEM_SHARED((TILES * 2 * NP,), jnp.float32),
            pltpu.SemaphoreType.DMA,
        ],
    )
    def k(fsrc_h, fdst_h, fis_h, fid_h, dst_h, attn_h, zeros_h,
          ex_out, den_out, sidx, didx, srows, drows, tp, attn_v,
          logits_t, ex_t, dst_t, den_t, cacc, ctmp, stage, sem):
        c = lax.axis_index("c")
        s = lax.axis_index("s")
        iota16 = lax.iota(jnp.int32, 16)
        ebase = s * EPT
        pltpu.sync_copy(attn_h.at[pl.ds(c * 512, 512)], attn_v)
        pltpu.sync_copy(dst_h.at[pl.ds(ebase, EPT)], dst_t)
        pltpu.sync_copy(zeros_h.at[pl.ds(0, 2 * EPT)], logits_t)
        pltpu.sync_copy(zeros_h, den_t)

        for j in range(4):
            jh = j // 2
            attn_vr = [attn_v[pl.ds(j * 128 + 16 * v, 16)] for v in range(8)]

            def chunk(i, carry, j=j, jh=jh, attn_vr=attn_vr):
                base = ebase + i * CH
                pltpu.sync_copy(fis_h.at[pl.ds((4 * c + j) * E + base, CH)], sidx)
                pltpu.sync_copy(fid_h.at[pl.ds((4 * c + j) * E + base, CH)], didx)
                d1 = pltpu.async_copy(fsrc_h.at[sidx], srows, sem)
                d2 = pltpu.async_copy(fdst_h.at[didx], drows, sem)
                d1.wait()
                d2.wait()

                def group(g, carry2):
                    def edge(e, carry3):
                        acc = None
                        for v in range(8):
                            sv = srows[g * 16 + e, pl.ds(16 * v, 16)]
                            dv = drows[g * 16 + e, pl.ds(16 * v, 16)]
                            u = sv + dv
                            lr = jnp.maximum(u, 0.2 * u)
                            t = lr * attn_vr[v]
                            acc = t if acc is None else acc + t
                        tp[pl.ds(e * 16, 16)] = acc
                        return carry3
                    lax.fori_loop(0, 16, edge, 0)
                    res = None
                    for c16 in range(16):
                        col = plsc.load_gather(tp, [iota16 * 16 + c16])
                        res = col if res is None else res + col
                    off = jh * EPT + i * CH + g * 16
                    logits_t[pl.ds(off, 16)] = logits_t[pl.ds(off, 16)] + res
                    return carry2
                lax.fori_loop(0, CH // 16, group, 0)
                return carry
            lax.fori_loop(0, EPT // CH, chunk, 0)

        for jh in range(2):
            def expgrp(g, carry, jh=jh):
                lv = logits_t[pl.ds(jh * EPT + g * 16, 16)]
                exv = jnp.exp(lv)
                ex_t[pl.ds(jh * EPT + g * 16, 16)] = exv
                dstv = dst_t[pl.ds(g * 16, 16)]
                plsc.addupdate_scatter(den_t, [dstv + jh * NP], exv)
                return carry
            lax.fori_loop(0, EPT // 16, expgrp, 0)
            pltpu.sync_copy(ex_t.at[pl.ds(jh * EPT, EPT)],
                            ex_out.at[pl.ds((c * 2 + jh) * E + ebase, EPT)])

        pltpu.sync_copy(den_t, stage.at[pl.ds(s * 2 * NP, 2 * NP)])
        plsc.subcore_barrier()
        STRIPE = 2 * NP // TILES
        for kk in range(TILES):
            pltpu.sync_copy(stage.at[pl.ds(kk * 2 * NP + s * STRIPE, STRIPE)],
                            ctmp)
            def addg(g, carry, kk=kk):
                if kk == 0:
                    cacc[pl.ds(g * 16, 16)] = ctmp[pl.ds(g * 16, 16)]
                else:
                    cacc[pl.ds(g * 16, 16)] = (cacc[pl.ds(g * 16, 16)]
                                               + ctmp[pl.ds(g * 16, 16)])
                return carry
            lax.fori_loop(0, STRIPE // 16, addg, 0)
        pltpu.sync_copy(cacc, den_out.at[pl.ds(c * 2 * NP + s * STRIPE, STRIPE)])

    return k(fsrc_t, fdst_t, fidx_src, fidx_dst, dst, attn_t, zeros_flat)


def _gat_agg_sc(fsrc_t, fidx_src, dst, ex, denom, zeros2d):
    mesh = plsc.VectorSubcoreMesh(
        core_axis_name="c", subcore_axis_name="s",
        num_cores=CORES, num_subcores=TILES)

    @functools.partial(
        pl.kernel,
        out_type=jax.ShapeDtypeStruct((8, NP, 128), jnp.float32),
        mesh=mesh,
        compiler_params=pltpu.CompilerParams(needs_layout_passes=False),
        scratch_types=[
            pltpu.VMEM((CH,), jnp.int32),
            pltpu.VMEM((CH,), jnp.int32),
            pltpu.VMEM((CH,), jnp.float32),
            pltpu.VMEM((CH,), jnp.float32),
            pltpu.VMEM((CH, 128), jnp.float32),
            pltpu.VMEM((2 * NP,), jnp.float32),
            pltpu.VMEM_SHARED((NP, 128), jnp.float32),
            pltpu.SemaphoreType.DMA,
        ],
    )
    def k(fsrc_h, fis_h, dst_h, ex_h, den_h, zeros_h, out_h,
          sidx, dstv, exb, alb, rows, den_t, acc, sem):
        c = lax.axis_index("c")
        s = lax.axis_index("s")
        ebase = s * EPT
        pltpu.sync_copy(den_h.at[pl.ds(c * 2 * NP, 2 * NP)], den_t)
        for j in range(4):
            jh = j // 2
            pltpu.sync_copy(zeros_h, acc.at[pl.ds(s * NPT, NPT)])
            plsc.subcore_barrier()

            def chunk(i, carry, j=j, jh=jh):
                base = ebase + i * CH
                pltpu.sync_copy(fis_h.at[pl.ds((4 * c + j) * E + base, CH)], sidx)
                pltpu.sync_copy(dst_h.at[pl.ds(base, CH)], dstv)
                pltpu.sync_copy(ex_h.at[pl.ds((c * 2 + jh) * E + base, CH)], exb)
                pltpu.async_copy(fsrc_h.at[sidx], rows, sem).wait()

                def alpha_grp(g, carry2, jh=jh):
                    dv = dstv[pl.ds(g * 16, 16)]
                    exv = exb[pl.ds(g * 16, 16)]
                    dnv = plsc.load_gather(den_t, [dv + jh * NP])
                    alb[pl.ds(g * 16, 16)] = exv / jnp.maximum(dnv, 1e-9)
                    return carry2
                lax.fori_loop(0, CH // 16, alpha_grp, 0)

                def scale_edge(e, carry2):
                    av = plsc.load_gather(alb, [jnp.full((16,), e, jnp.int32)])
                    for v in range(8):
                        rows[e, pl.ds(16 * v, 16)] = (
                            rows[e, pl.ds(16 * v, 16)] * av)
                    return carry2
                lax.fori_loop(0, CH, scale_edge, 0)
                pltpu.sync_copy(rows, acc.at[dstv], add=True)
                return carry
            lax.fori_loop(0, EPT // CH, chunk, 0)
            plsc.subcore_barrier()
            pltpu.sync_copy(acc.at[pl.ds(s * NPT, NPT)],
                            out_h.at[4 * c + j, pl.ds(s * NPT, NPT)])
            plsc.subcore_barrier()

    return k(fsrc_t, fidx_src, dst, ex, denom, zeros2d)


def kernel(x, edge_index, edge_type, ggc_W, ggc_b, gru_Wih, gru_Whh, gru_bih, gru_bhh,
           gat_Wsrc, gat_bsrc, gat_Wdst, gat_bdst, gat_attn, gat_bias,
           skip_W, skip_b, bn1_g, bn1_b, bn2_g, bn2_b,
           red_W, red_b, gate_W, gate_b, cls_W1, cls_b1, cls_W2, cls_b2):
    src = edge_index[0]
    dst = edge_index[1]
    zeros = jnp.zeros((NPT, 128), jnp.float32)
    eidx = edge_type.astype(jnp.int32) * NP + src
    eidx2 = jnp.concatenate([eidx, eidx + NE * NP])

    h = jnp.pad(x, ((0, NP - N), (0, H - IN)))
    for _ in range(NS):
        proj = jnp.einsum('ni,toi->tno', h, ggc_W) + ggc_b[:, None, :]
        table = proj.reshape(NE, NP, 2, 128).transpose(2, 0, 1, 3).reshape(-1, 128)
        a2 = _ggc_edge_sc(table, eidx2, dst, zeros)
        a = jnp.concatenate([a2[0, :N], a2[1, :N]], axis=1)
        hh = h[:N]
        gi = a @ gru_Wih.T + gru_bih
        gh = hh @ gru_Whh.T + gru_bhh
        i_r, i_z, i_n = jnp.split(gi, 3, axis=1)
        h_r, h_z, h_n = jnp.split(gh, 3, axis=1)
        r = jax.nn.sigmoid(i_r + h_r)
        z = jax.nn.sigmoid(i_z + h_z)
        n = jnp.tanh(i_n + r * h_n)
        hnew = (1.0 - z) * n + z * hh
        h = jnp.pad(hnew, ((0, NP - N), (0, 0)))
    h = h[:N]
    h2 = jax.nn.elu(h)
    h2 = (h2 - h2.mean(0)) / jnp.sqrt(h2.var(0) + 1e-5) * bn2_g + bn2_b
    fsrc = h2 @ gat_Wsrc.T + gat_bsrc
    fdst = h2 @ gat_Wdst.T + gat_bdst
    fsrc_t = fsrc.reshape(N, 8, 128).transpose(1, 0, 2).reshape(8 * N, 128)
    fdst_t = fdst.reshape(N, 8, 128).transpose(1, 0, 2).reshape(8 * N, 128)
    soff = (jnp.arange(8, dtype=jnp.int32) * N)[:, None]
    fidx_src = (soff + src[None, :]).reshape(-1)
    fidx_dst = (soff + dst[None, :]).reshape(-1)
    attn_t = gat_attn.reshape(-1)
    zeros_flat = jnp.zeros((2 * NP,), jnp.float32)
    ex, denom = _gat_logits_sc(fsrc_t, fdst_t, fidx_src, fidx_dst, dst,
                               attn_t, zeros_flat)
    gat8 = _gat_agg_sc(fsrc_t, fidx_src, dst, ex, denom, zeros)
    out = (gat8.reshape(4, 2, NP, 128)[:, :, :N]
           .transpose(2, 0, 1, 3).reshape(N, HEADS, H))
    out = out + gat_bias.reshape(1, HEADS, H)
    h1 = jax.nn.elu(out).reshape(N, HEADS * H)
    skip = x @ skip_W.T + skip_b
    h1 = h1 + skip
    h1 = (h1 - h1.mean(0)) / jnp.sqrt(h1.var(0) + 1e-5) * bn1_g + bn1_b
    hr = h1 @ red_W.T + red_b
    gate = hr @ gate_W.T + gate_b
    gate = jax.nn.softmax(gate, axis=0)
    h_g = (gate * hr).sum(0, keepdims=True)
    z1 = jax.nn.relu(h_g @ cls_W1.T + cls_b1)
    return z1 @ cls_W2.T + cls_b2


# speedup vs baseline: 5.8111x; 1.0870x over previous
import functools

import jax
import jax.numpy as jnp
from jax import lax
from jax.experimental import pallas as pl
from jax.experimental.pallas import tpu as pltpu
from jax.experimental.pallas import tpu_sc as plsc

N = 10000
E = 160000
IN = 128
H = 256
HEADS = 4
NCLS = 10
NE = 4
NS = 3

NP = 10240
CH = 80
TILES = 16
CORES = 2
EPT = E // TILES
NPT = NP // TILES


def _ggc_edge_sc(table, eidx2, dst, zeros):
    mesh = plsc.VectorSubcoreMesh(
        core_axis_name="c", subcore_axis_name="s",
        num_cores=CORES, num_subcores=TILES)

    @functools.partial(
        pl.kernel,
        out_type=jax.ShapeDtypeStruct((CORES, NP, 128), jnp.float32),
        mesh=mesh,
        compiler_params=pltpu.CompilerParams(needs_layout_passes=False),
        scratch_types=[
            pltpu.VMEM((CH,), jnp.int32),
            pltpu.VMEM((CH,), jnp.int32),
            pltpu.VMEM((CH, 128), jnp.float32),
            pltpu.VMEM((CH,), jnp.int32),
            pltpu.VMEM((CH,), jnp.int32),
            pltpu.VMEM((CH, 128), jnp.float32),
            pltpu.VMEM_SHARED((NP, 128), jnp.float32),
            pltpu.SemaphoreType.DMA,
            pltpu.SemaphoreType.DMA,
        ],
    )
    def k(table_h, eidx_h, dst_h, zeros_h, out_h,
          idx_a, dst_a, rows_a, idx_b, dst_b, rows_b, acc, sem_a, sem_b):
        c = lax.axis_index("c")
        s = lax.axis_index("s")
        pltpu.sync_copy(zeros_h, acc.at[pl.ds(s * NPT, NPT)])
        plsc.subcore_barrier()

        def copy_idx(i, idx_v, dst_v):
            base = s * EPT + i * CH
            pltpu.sync_copy(eidx_h.at[pl.ds(c * E + base, CH)], idx_v)
            pltpu.sync_copy(dst_h.at[pl.ds(base, CH)], dst_v)

        copy_idx(0, idx_a, dst_a)
        pltpu.async_copy(table_h.at[idx_a], rows_a, sem_a)

        def pair(i, carry):
            copy_idx(2 * i + 1, idx_b, dst_b)
            pltpu.async_copy(table_h.at[idx_b], rows_b, sem_b)
            pltpu.make_async_copy(table_h.at[idx_a], rows_a, sem_a).wait()
            pltpu.sync_copy(rows_a, acc.at[dst_a], add=True)
            copy_idx(2 * i + 2, idx_a, dst_a)
            pltpu.async_copy(table_h.at[idx_a], rows_a, sem_a)
            pltpu.make_async_copy(table_h.at[idx_b], rows_b, sem_b).wait()
            pltpu.sync_copy(rows_b, acc.at[dst_b], add=True)
            return carry

        lax.fori_loop(0, (EPT // CH) // 2, pair, 0)
        pltpu.make_async_copy(table_h.at[idx_a], rows_a, sem_a).wait()
        pltpu.sync_copy(rows_a, acc.at[dst_a], add=True)
        plsc.subcore_barrier()
        pltpu.sync_copy(acc.at[pl.ds(s * NPT, NPT)],
                        out_h.at[c, pl.ds(s * NPT, NPT)])

    return k(table, eidx2, dst, zeros)


def _gat_logits_sc(fsrc_t, fdst_t, fidx_src, fidx_dst, dst, attn_t, zeros_flat):
    mesh = plsc.VectorSubcoreMesh(
        core_axis_name="c", subcore_axis_name="s",
        num_cores=CORES, num_subcores=TILES)
    @functools.partial(
        pl.kernel,
        out_type=(jax.ShapeDtypeStruct((4 * E,), jnp.float32),
                  jax.ShapeDtypeStruct((4 * NP,), jnp.float32)),
        mesh=mesh,
        compiler_params=pltpu.CompilerParams(needs_layout_passes=False),
        scratch_types=[
            pltpu.VMEM((CH,), jnp.int32),
            pltpu.VMEM((CH,), jnp.int32),
            pltpu.VMEM((CH, 128), jnp.float32),
            pltpu.VMEM((CH, 128), jnp.float32),
            pltpu.VMEM((256,), jnp.float32),
            pltpu.VMEM((512,), jnp.float32),
            pltpu.VMEM((2 * EPT,), jnp.float32),
            pltpu.VMEM((2 * EPT,), jnp.float32),
            pltpu.VMEM((EPT,), jnp.int32),
            pltpu.VMEM((2 * NP,), jnp.float32),
            pltpu.VMEM((1280,), jnp.float32),
            pltpu.VMEM((1280,), jnp.float32),
            pltpu.VMEM_SHARED((TILES * 2 * NP,), jnp.float32),
            pltpu.SemaphoreType.DMA,
        ],
    )
    def k(fsrc_h, fdst_h, fis_h, fid_h, dst_h, attn_h, zeros_h,
          ex_out, den_out, sidx, didx, srows, drows, tp, attn_v,
          logits_t, ex_t, dst_t, den_t, cacc, ctmp, stage, sem):
        c = lax.axis_index("c")
        s = lax.axis_index("s")
        iota16 = lax.iota(jnp.int32, 16)
        ebase = s * EPT
        pltpu.sync_copy(attn_h.at[pl.ds(c * 512, 512)], attn_v)
        pltpu.sync_copy(dst_h.at[pl.ds(ebase, EPT)], dst_t)
        pltpu.sync_copy(zeros_h.at[pl.ds(0, 2 * EPT)], logits_t)
        pltpu.sync_copy(zeros_h, den_t)

        for j in range(4):
            jh = j // 2
            attn_vr = [attn_v[pl.ds(j * 128 + 16 * v, 16)] for v in range(8)]

            def chunk(i, carry, j=j, jh=jh, attn_vr=attn_vr):
                base = ebase + i * CH
                pltpu.sync_copy(fis_h.at[pl.ds((4 * c + j) * E + base, CH)], sidx)
                pltpu.sync_copy(fid_h.at[pl.ds((4 * c + j) * E + base, CH)], didx)
                d1 = pltpu.async_copy(fsrc_h.at[sidx], srows, sem)
                d2 = pltpu.async_copy(fdst_h.at[didx], drows, sem)
                d1.wait()
                d2.wait()

                def group(g, carry2):
                    for e in range(16):
                        acc = None
                        for v in range(8):
                            sv = srows[g * 16 + e, pl.ds(16 * v, 16)]
                            dv = drows[g * 16 + e, pl.ds(16 * v, 16)]
                            u = sv + dv
                            lr = jnp.maximum(u, 0.2 * u)
                            t = lr * attn_vr[v]
                            acc = t if acc is None else acc + t
                        tp[pl.ds(e * 16, 16)] = acc
                    res = None
                    for c16 in range(16):
                        col = plsc.load_gather(tp, [iota16 * 16 + c16])
                        res = col if res is None else res + col
                    off = jh * EPT + i * CH + g * 16
                    logits_t[pl.ds(off, 16)] = logits_t[pl.ds(off, 16)] + res
                    return carry2
                lax.fori_loop(0, CH // 16, group, 0)
                return carry
            lax.fori_loop(0, EPT // CH, chunk, 0)

        for jh in range(2):
            def expgrp(g, carry, jh=jh):
                lv = logits_t[pl.ds(jh * EPT + g * 16, 16)]
                exv = jnp.exp(lv)
                ex_t[pl.ds(jh * EPT + g * 16, 16)] = exv
                dstv = dst_t[pl.ds(g * 16, 16)]
                plsc.addupdate_scatter(den_t, [dstv + jh * NP], exv)
                return carry
            lax.fori_loop(0, EPT // 16, expgrp, 0)
            pltpu.sync_copy(ex_t.at[pl.ds(jh * EPT, EPT)],
                            ex_out.at[pl.ds((c * 2 + jh) * E + ebase, EPT)])

        pltpu.sync_copy(den_t, stage.at[pl.ds(s * 2 * NP, 2 * NP)])
        plsc.subcore_barrier()
        STRIPE = 2 * NP // TILES
        for kk in range(TILES):
            pltpu.sync_copy(stage.at[pl.ds(kk * 2 * NP + s * STRIPE, STRIPE)],
                            ctmp)
            def addg(g, carry, kk=kk):
                if kk == 0:
                    cacc[pl.ds(g * 16, 16)] = ctmp[pl.ds(g * 16, 16)]
                else:
                    cacc[pl.ds(g * 16, 16)] = (cacc[pl.ds(g * 16, 16)]
                                               + ctmp[pl.ds(g * 16, 16)])
                return carry
            lax.fori_loop(0, STRIPE // 16, addg, 0)
        pltpu.sync_copy(cacc, den_out.at[pl.ds(c * 2 * NP + s * STRIPE, STRIPE)])

    return k(fsrc_t, fdst_t, fidx_src, fidx_dst, dst, attn_t, zeros_flat)


def _gat_agg_sc(fsrc_t, fidx_src, dst, ex, denom, zeros2d):
    mesh = plsc.VectorSubcoreMesh(
        core_axis_name="c", subcore_axis_name="s",
        num_cores=CORES, num_subcores=TILES)

    @functools.partial(
        pl.kernel,
        out_type=jax.ShapeDtypeStruct((8, NP, 128), jnp.float32),
        mesh=mesh,
        compiler_params=pltpu.CompilerParams(needs_layout_passes=False),
        scratch_types=[
            pltpu.VMEM((CH,), jnp.int32),
            pltpu.VMEM((CH,), jnp.int32),
            pltpu.VMEM((CH,), jnp.float32),
            pltpu.VMEM((CH,), jnp.float32),
            pltpu.VMEM((CH, 128), jnp.float32),
            pltpu.VMEM((2 * NP,), jnp.float32),
            pltpu.VMEM_SHARED((NP, 128), jnp.float32),
            pltpu.SemaphoreType.DMA,
        ],
    )
    def k(fsrc_h, fis_h, dst_h, ex_h, den_h, zeros_h, out_h,
          sidx, dstv, exb, alb, rows, den_t, acc, sem):
        c = lax.axis_index("c")
        s = lax.axis_index("s")
        ebase = s * EPT
        pltpu.sync_copy(den_h.at[pl.ds(c * 2 * NP, 2 * NP)], den_t)
        for j in range(4):
            jh = j // 2
            pltpu.sync_copy(zeros_h, acc.at[pl.ds(s * NPT, NPT)])
            plsc.subcore_barrier()

            def chunk(i, carry, j=j, jh=jh):
                base = ebase + i * CH
                pltpu.sync_copy(fis_h.at[pl.ds((4 * c + j) * E + base, CH)], sidx)
                pltpu.sync_copy(dst_h.at[pl.ds(base, CH)], dstv)
                pltpu.sync_copy(ex_h.at[pl.ds((c * 2 + jh) * E + base, CH)], exb)
                pltpu.async_copy(fsrc_h.at[sidx], rows, sem).wait()

                def scale_grp(g, carry2, jh=jh):
                    dv = dstv[pl.ds(g * 16, 16)]
                    exv = exb[pl.ds(g * 16, 16)]
                    dnv = plsc.load_gather(den_t, [dv + jh * NP])
                    alb[pl.ds(g * 16, 16)] = exv / jnp.maximum(dnv, 1e-9)
                    for e in range(16):
                        av = plsc.load_gather(
                            alb, [g * 16 + e + jnp.zeros((16,), jnp.int32)])
                        row = g * 16 + e
                        for v in range(8):
                            rows[row, pl.ds(16 * v, 16)] = (
                                rows[row, pl.ds(16 * v, 16)] * av)
                    return carry2
                lax.fori_loop(0, CH // 16, scale_grp, 0)
                pltpu.sync_copy(rows, acc.at[dstv], add=True)
                return carry
            lax.fori_loop(0, EPT // CH, chunk, 0)
            plsc.subcore_barrier()
            pltpu.sync_copy(acc.at[pl.ds(s * NPT, NPT)],
                            out_h.at[4 * c + j, pl.ds(s * NPT, NPT)])
            plsc.subcore_barrier()

    return k(fsrc_t, fidx_src, dst, ex, denom, zeros2d)


def kernel(x, edge_index, edge_type, ggc_W, ggc_b, gru_Wih, gru_Whh, gru_bih, gru_bhh,
           gat_Wsrc, gat_bsrc, gat_Wdst, gat_bdst, gat_attn, gat_bias,
           skip_W, skip_b, bn1_g, bn1_b, bn2_g, bn2_b,
           red_W, red_b, gate_W, gate_b, cls_W1, cls_b1, cls_W2, cls_b2):
    src = edge_index[0]
    dst = edge_index[1]
    zeros = jnp.zeros((NPT, 128), jnp.float32)
    eidx = edge_type.astype(jnp.int32) * NP + src
    eidx2 = jnp.concatenate([eidx, eidx + NE * NP])

    h = jnp.pad(x, ((0, NP - N), (0, H - IN)))
    for _ in range(NS):
        proj = jnp.einsum('ni,toi->tno', h, ggc_W) + ggc_b[:, None, :]
        table = proj.reshape(NE, NP, 2, 128).transpose(2, 0, 1, 3).reshape(-1, 128)
        a2 = _ggc_edge_sc(table, eidx2, dst, zeros)
        a = jnp.concatenate([a2[0, :N], a2[1, :N]], axis=1)
        hh = h[:N]
        gi = a @ gru_Wih.T + gru_bih
        gh = hh @ gru_Whh.T + gru_bhh
        i_r, i_z, i_n = jnp.split(gi, 3, axis=1)
        h_r, h_z, h_n = jnp.split(gh, 3, axis=1)
        r = jax.nn.sigmoid(i_r + h_r)
        z = jax.nn.sigmoid(i_z + h_z)
        n = jnp.tanh(i_n + r * h_n)
        hnew = (1.0 - z) * n + z * hh
        h = jnp.pad(hnew, ((0, NP - N), (0, 0)))
    h = h[:N]
    h2 = jax.nn.elu(h)
    h2 = (h2 - h2.mean(0)) / jnp.sqrt(h2.var(0) + 1e-5) * bn2_g + bn2_b
    fsrc = h2 @ gat_Wsrc.T + gat_bsrc
    fdst = h2 @ gat_Wdst.T + gat_bdst
    fsrc_t = fsrc.reshape(N, 8, 128).transpose(1, 0, 2).reshape(8 * N, 128)
    fdst_t = fdst.reshape(N, 8, 128).transpose(1, 0, 2).reshape(8 * N, 128)
    soff = (jnp.arange(8, dtype=jnp.int32) * N)[:, None]
    fidx_src = (soff + src[None, :]).reshape(-1)
    fidx_dst = (soff + dst[None, :]).reshape(-1)
    attn_t = gat_attn.reshape(-1)
    zeros_flat = jnp.zeros((2 * NP,), jnp.float32)
    ex, denom = _gat_logits_sc(fsrc_t, fdst_t, fidx_src, fidx_dst, dst,
                               attn_t, zeros_flat)
    gat8 = _gat_agg_sc(fsrc_t, fidx_src, dst, ex, denom, zeros)
    out = (gat8.reshape(4, 2, NP, 128)[:, :, :N]
           .transpose(2, 0, 1, 3).reshape(N, HEADS, H))
    out = out + gat_bias.reshape(1, HEADS, H)
    h1 = jax.nn.elu(out).reshape(N, HEADS * H)
    skip = x @ skip_W.T + skip_b
    h1 = h1 + skip
    h1 = (h1 - h1.mean(0)) / jnp.sqrt(h1.var(0) + 1e-5) * bn1_g + bn1_b
    hr = h1 @ red_W.T + red_b
    gate = hr @ gate_W.T + gate_b
    gate = jax.nn.softmax(gate, axis=0)
    h_g = (gate * hr).sum(0, keepdims=True)
    z1 = jax.nn.relu(h_g @ cls_W1.T + cls_b1)
    return z1 @ cls_W2.T + cls_b2


# speedup vs baseline: 7.6067x; 1.3090x over previous
import functools

import jax
import jax.numpy as jnp
from jax import lax
from jax.experimental import pallas as pl
from jax.experimental.pallas import tpu as pltpu
from jax.experimental.pallas import tpu_sc as plsc

N = 10000
E = 160000
IN = 128
H = 256
HEADS = 4
NCLS = 10
NE = 4
NS = 3

NP = 10240
CH = 80
TILES = 16
CORES = 2
EPT = E // TILES
NPT = NP // TILES


def _ggc_edge_sc(table, eidx2, dst, zeros):
    mesh = plsc.VectorSubcoreMesh(
        core_axis_name="c", subcore_axis_name="s",
        num_cores=CORES, num_subcores=TILES)

    @functools.partial(
        pl.kernel,
        out_type=jax.ShapeDtypeStruct((CORES, NP, 128), jnp.float32),
        mesh=mesh,
        compiler_params=pltpu.CompilerParams(needs_layout_passes=False),
        scratch_types=[
            pltpu.VMEM((CH,), jnp.int32),
            pltpu.VMEM((CH,), jnp.int32),
            pltpu.VMEM((CH, 128), jnp.float32),
            pltpu.VMEM((CH,), jnp.int32),
            pltpu.VMEM((CH,), jnp.int32),
            pltpu.VMEM((CH, 128), jnp.float32),
            pltpu.VMEM_SHARED((NP, 128), jnp.float32),
            pltpu.SemaphoreType.DMA,
            pltpu.SemaphoreType.DMA,
        ],
    )
    def k(table_h, eidx_h, dst_h, zeros_h, out_h,
          idx_a, dst_a, rows_a, idx_b, dst_b, rows_b, acc, sem_a, sem_b):
        c = lax.axis_index("c")
        s = lax.axis_index("s")
        pltpu.sync_copy(zeros_h, acc.at[pl.ds(s * NPT, NPT)])
        plsc.subcore_barrier()

        def copy_idx(i, idx_v, dst_v):
            base = s * EPT + i * CH
            pltpu.sync_copy(eidx_h.at[pl.ds(c * E + base, CH)], idx_v)
            pltpu.sync_copy(dst_h.at[pl.ds(base, CH)], dst_v)

        copy_idx(0, idx_a, dst_a)
        pltpu.async_copy(table_h.at[idx_a], rows_a, sem_a)

        def pair(i, carry):
            copy_idx(2 * i + 1, idx_b, dst_b)
            pltpu.async_copy(table_h.at[idx_b], rows_b, sem_b)
            pltpu.make_async_copy(table_h.at[idx_a], rows_a, sem_a).wait()
            pltpu.sync_copy(rows_a, acc.at[dst_a], add=True)
            copy_idx(2 * i + 2, idx_a, dst_a)
            pltpu.async_copy(table_h.at[idx_a], rows_a, sem_a)
            pltpu.make_async_copy(table_h.at[idx_b], rows_b, sem_b).wait()
            pltpu.sync_copy(rows_b, acc.at[dst_b], add=True)
            return carry

        lax.fori_loop(0, (EPT // CH) // 2, pair, 0)
        pltpu.make_async_copy(table_h.at[idx_a], rows_a, sem_a).wait()
        pltpu.sync_copy(rows_a, acc.at[dst_a], add=True)
        plsc.subcore_barrier()
        pltpu.sync_copy(acc.at[pl.ds(s * NPT, NPT)],
                        out_h.at[c, pl.ds(s * NPT, NPT)])

    return k(table, eidx2, dst, zeros)


def _gat_logits_sc(fsrc_t, fdst_t, fidx_src, fidx_dst, dst, attn_t, zeros_flat):
    mesh = plsc.VectorSubcoreMesh(
        core_axis_name="c", subcore_axis_name="s",
        num_cores=CORES, num_subcores=TILES)
    @functools.partial(
        pl.kernel,
        out_type=(jax.ShapeDtypeStruct((4 * E,), jnp.float32),
                  jax.ShapeDtypeStruct((4 * NP,), jnp.float32)),
        mesh=mesh,
        compiler_params=pltpu.CompilerParams(needs_layout_passes=False),
        scratch_types=[
            pltpu.VMEM((CH,), jnp.int32),
            pltpu.VMEM((CH,), jnp.int32),
            pltpu.VMEM((CH, 128), jnp.float32),
            pltpu.VMEM((CH, 128), jnp.float32),
            pltpu.VMEM((CH,), jnp.int32),
            pltpu.VMEM((CH,), jnp.int32),
            pltpu.VMEM((CH, 128), jnp.float32),
            pltpu.VMEM((CH, 128), jnp.float32),
            pltpu.VMEM((256,), jnp.float32),
            pltpu.VMEM((512,), jnp.float32),
            pltpu.VMEM((2 * EPT,), jnp.float32),
            pltpu.VMEM((EPT,), jnp.int32),
            pltpu.VMEM((2 * NP,), jnp.float32),
            pltpu.VMEM((1280,), jnp.float32),
            pltpu.VMEM((1280,), jnp.float32),
            pltpu.VMEM_SHARED((TILES * 2 * NP,), jnp.float32),
            pltpu.SemaphoreType.DMA,
            pltpu.SemaphoreType.DMA,
            pltpu.SemaphoreType.DMA,
            pltpu.SemaphoreType.DMA,
        ],
    )
    def k(fsrc_h, fdst_h, fis_h, fid_h, dst_h, attn_h, zeros_h,
          ex_out, den_out, sidx_a, didx_a, srows_a, drows_a,
          sidx_b, didx_b, srows_b, drows_b, tp, attn_v,
          logits_t, dst_t, den_t, cacc, ctmp, stage,
          sem_sa, sem_da, sem_sb, sem_db):
        c = lax.axis_index("c")
        s = lax.axis_index("s")
        iota16 = lax.iota(jnp.int32, 16)
        ebase = s * EPT
        NCH = EPT // CH
        pltpu.sync_copy(attn_h.at[pl.ds(c * 512, 512)], attn_v)
        pltpu.sync_copy(dst_h.at[pl.ds(ebase, EPT)], dst_t)
        pltpu.sync_copy(zeros_h.at[pl.ds(0, 2 * EPT)], logits_t)
        pltpu.sync_copy(zeros_h, den_t)

        def fetch(kc, sidx, didx, srows, drows, sem_s, sem_d):
            j = kc // NCH
            ii = kc % NCH
            off = (4 * c + j) * E + ebase + ii * CH
            pltpu.sync_copy(fis_h.at[pl.ds(off, CH)], sidx)
            pltpu.sync_copy(fid_h.at[pl.ds(off, CH)], didx)
            pltpu.async_copy(fsrc_h.at[sidx], srows, sem_s)
            pltpu.async_copy(fdst_h.at[didx], drows, sem_d)

        def wait_set(sidx, srows, drows, sem_s, sem_d):
            pltpu.make_async_copy(fsrc_h.at[sidx], srows, sem_s).wait()
            pltpu.make_async_copy(fsrc_h.at[sidx], drows, sem_d).wait()

        def compute(kc, srows, drows):
            j = kc // NCH
            jh = kc // (2 * NCH)
            ii = kc % NCH
            attn_vr = [attn_v[pl.ds(j * 128 + 16 * v, 16)] for v in range(8)]

            def group(g, carry2):
                for e in range(16):
                    acc = None
                    for v in range(8):
                        sv = srows[g * 16 + e, pl.ds(16 * v, 16)]
                        dv = drows[g * 16 + e, pl.ds(16 * v, 16)]
                        u = sv + dv
                        lr = jnp.maximum(u, 0.2 * u)
                        t = lr * attn_vr[v]
                        acc = t if acc is None else acc + t
                    tp[pl.ds(e * 16, 16)] = acc
                res = None
                for c16 in range(16):
                    col = plsc.load_gather(tp, [iota16 * 16 + c16])
                    res = col if res is None else res + col
                off = jh * EPT + ii * CH + g * 16
                logits_t[pl.ds(off, 16)] = logits_t[pl.ds(off, 16)] + res
                return carry2
            lax.fori_loop(0, CH // 16, group, 0)

        NPAIR = 4 * NCH // 2
        fetch(0, sidx_a, didx_a, srows_a, drows_a, sem_sa, sem_da)

        def pair(p, carry):
            fetch(2 * p + 1, sidx_b, didx_b, srows_b, drows_b, sem_sb, sem_db)
            wait_set(sidx_a, srows_a, drows_a, sem_sa, sem_da)
            compute(2 * p, srows_a, drows_a)

            @pl.when(p + 1 < NPAIR)
            def _():
                fetch(2 * p + 2, sidx_a, didx_a, srows_a, drows_a,
                      sem_sa, sem_da)
            wait_set(sidx_b, srows_b, drows_b, sem_sb, sem_db)
            compute(2 * p + 1, srows_b, drows_b)
            return carry
        lax.fori_loop(0, NPAIR, pair, 0)

        for jh in range(2):
            def expgrp(g, carry, jh=jh):
                lv = logits_t[pl.ds(jh * EPT + g * 16, 16)]
                exv = jnp.exp(lv)
                logits_t[pl.ds(jh * EPT + g * 16, 16)] = exv
                dstv = dst_t[pl.ds(g * 16, 16)]
                plsc.addupdate_scatter(den_t, [dstv + jh * NP], exv)
                return carry
            lax.fori_loop(0, EPT // 16, expgrp, 0)
            pltpu.sync_copy(logits_t.at[pl.ds(jh * EPT, EPT)],
                            ex_out.at[pl.ds((c * 2 + jh) * E + ebase, EPT)])

        pltpu.sync_copy(den_t, stage.at[pl.ds(s * 2 * NP, 2 * NP)])
        plsc.subcore_barrier()
        STRIPE = 2 * NP // TILES
        for kk in range(TILES):
            pltpu.sync_copy(stage.at[pl.ds(kk * 2 * NP + s * STRIPE, STRIPE)],
                            ctmp)
            def addg(g, carry, kk=kk):
                if kk == 0:
                    cacc[pl.ds(g * 16, 16)] = ctmp[pl.ds(g * 16, 16)]
                else:
                    cacc[pl.ds(g * 16, 16)] = (cacc[pl.ds(g * 16, 16)]
                                               + ctmp[pl.ds(g * 16, 16)])
                return carry
            lax.fori_loop(0, STRIPE // 16, addg, 0)
        pltpu.sync_copy(cacc, den_out.at[pl.ds(c * 2 * NP + s * STRIPE, STRIPE)])

    return k(fsrc_t, fdst_t, fidx_src, fidx_dst, dst, attn_t, zeros_flat)


def _gat_agg_sc(fsrc_t, fidx_src, dst, ex, denom, zeros2d):
    mesh = plsc.VectorSubcoreMesh(
        core_axis_name="c", subcore_axis_name="s",
        num_cores=CORES, num_subcores=TILES)

    @functools.partial(
        pl.kernel,
        out_type=jax.ShapeDtypeStruct((8, NP, 128), jnp.float32),
        mesh=mesh,
        compiler_params=pltpu.CompilerParams(needs_layout_passes=False),
        scratch_types=[
            pltpu.VMEM((CH,), jnp.int32),
            pltpu.VMEM((CH,), jnp.int32),
            pltpu.VMEM((CH,), jnp.float32),
            pltpu.VMEM((CH, 128), jnp.float32),
            pltpu.VMEM((CH,), jnp.int32),
            pltpu.VMEM((CH,), jnp.int32),
            pltpu.VMEM((CH,), jnp.float32),
            pltpu.VMEM((CH, 128), jnp.float32),
            pltpu.VMEM((CH,), jnp.float32),
            pltpu.VMEM((2 * NP,), jnp.float32),
            pltpu.VMEM_SHARED((NP, 128), jnp.float32),
            pltpu.SemaphoreType.DMA,
            pltpu.SemaphoreType.DMA,
        ],
    )
    def k(fsrc_h, fis_h, dst_h, ex_h, den_h, zeros_h, out_h,
          sidx_a, dstv_a, exb_a, rows_a, sidx_b, dstv_b, exb_b, rows_b,
          alb, den_t, acc, sem_a, sem_b):
        c = lax.axis_index("c")
        s = lax.axis_index("s")
        ebase = s * EPT
        NCH = EPT // CH
        pltpu.sync_copy(den_h.at[pl.ds(c * 2 * NP, 2 * NP)], den_t)
        for j in range(4):
            jh = j // 2
            pltpu.sync_copy(zeros_h, acc.at[pl.ds(s * NPT, NPT)])
            plsc.subcore_barrier()

            def fetch(i, sidx, dstv, exb, rows, sem, j=j, jh=jh):
                base = ebase + i * CH
                pltpu.sync_copy(fis_h.at[pl.ds((4 * c + j) * E + base, CH)],
                                sidx)
                pltpu.sync_copy(dst_h.at[pl.ds(base, CH)], dstv)
                pltpu.sync_copy(ex_h.at[pl.ds((c * 2 + jh) * E + base, CH)],
                                exb)
                pltpu.async_copy(fsrc_h.at[sidx], rows, sem)

            def proc(sidx, dstv, exb, rows, sem, jh=jh):
                pltpu.make_async_copy(fsrc_h.at[sidx], rows, sem).wait()

                def scale_grp(g, carry2):
                    dv = dstv[pl.ds(g * 16, 16)]
                    exv = exb[pl.ds(g * 16, 16)]
                    dnv = plsc.load_gather(den_t, [dv + jh * NP])
                    alb[pl.ds(g * 16, 16)] = exv / jnp.maximum(dnv, 1e-9)
                    for e in range(16):
                        av = plsc.load_gather(
                            alb, [g * 16 + e + jnp.zeros((16,), jnp.int32)])
                        row = g * 16 + e
                        for v in range(8):
                            rows[row, pl.ds(16 * v, 16)] = (
                                rows[row, pl.ds(16 * v, 16)] * av)
                    return carry2
                lax.fori_loop(0, CH // 16, scale_grp, 0)
                pltpu.sync_copy(rows, acc.at[dstv], add=True)

            fetch(0, sidx_a, dstv_a, exb_a, rows_a, sem_a)

            def pair(p, carry):
                fetch(2 * p + 1, sidx_b, dstv_b, exb_b, rows_b, sem_b)
                proc(sidx_a, dstv_a, exb_a, rows_a, sem_a)
                fetch(2 * p + 2, sidx_a, dstv_a, exb_a, rows_a, sem_a)
                proc(sidx_b, dstv_b, exb_b, rows_b, sem_b)
                return carry
            lax.fori_loop(0, NCH // 2, pair, 0)
            proc(sidx_a, dstv_a, exb_a, rows_a, sem_a)
            plsc.subcore_barrier()
            pltpu.sync_copy(acc.at[pl.ds(s * NPT, NPT)],
                            out_h.at[4 * c + j, pl.ds(s * NPT, NPT)])
            plsc.subcore_barrier()

    return k(fsrc_t, fidx_src, dst, ex, denom, zeros2d)


def kernel(x, edge_index, edge_type, ggc_W, ggc_b, gru_Wih, gru_Whh, gru_bih, gru_bhh,
           gat_Wsrc, gat_bsrc, gat_Wdst, gat_bdst, gat_attn, gat_bias,
           skip_W, skip_b, bn1_g, bn1_b, bn2_g, bn2_b,
           red_W, red_b, gate_W, gate_b, cls_W1, cls_b1, cls_W2, cls_b2):
    src = edge_index[0]
    dst = edge_index[1]
    zeros = jnp.zeros((NPT, 128), jnp.float32)
    eidx = edge_type.astype(jnp.int32) * NP + src
    eidx2 = jnp.concatenate([eidx, eidx + NE * NP])

    h = jnp.pad(x, ((0, NP - N), (0, H - IN)))
    for _ in range(NS):
        proj = jnp.einsum('ni,toi->tno', h, ggc_W) + ggc_b[:, None, :]
        table = proj.reshape(NE, NP, 2, 128).transpose(2, 0, 1, 3).reshape(-1, 128)
        a2 = _ggc_edge_sc(table, eidx2, dst, zeros)
        a = jnp.concatenate([a2[0, :N], a2[1, :N]], axis=1)
        hh = h[:N]
        gi = a @ gru_Wih.T + gru_bih
        gh = hh @ gru_Whh.T + gru_bhh
        i_r, i_z, i_n = jnp.split(gi, 3, axis=1)
        h_r, h_z, h_n = jnp.split(gh, 3, axis=1)
        r = jax.nn.sigmoid(i_r + h_r)
        z = jax.nn.sigmoid(i_z + h_z)
        n = jnp.tanh(i_n + r * h_n)
        hnew = (1.0 - z) * n + z * hh
        h = jnp.pad(hnew, ((0, NP - N), (0, 0)))
    h = h[:N]
    h2 = jax.nn.elu(h)
    h2 = (h2 - h2.mean(0)) / jnp.sqrt(h2.var(0) + 1e-5) * bn2_g + bn2_b
    fsrc = h2 @ gat_Wsrc.T + gat_bsrc
    fdst = h2 @ gat_Wdst.T + gat_bdst
    fsrc_t = fsrc.reshape(N, 8, 128).transpose(1, 0, 2).reshape(8 * N, 128)
    fdst_t = fdst.reshape(N, 8, 128).transpose(1, 0, 2).reshape(8 * N, 128)
    soff = (jnp.arange(8, dtype=jnp.int32) * N)[:, None]
    fidx_src = (soff + src[None, :]).reshape(-1)
    fidx_dst = (soff + dst[None, :]).reshape(-1)
    attn_t = gat_attn.reshape(-1)
    zeros_flat = jnp.zeros((2 * NP,), jnp.float32)
    ex, denom = _gat_logits_sc(fsrc_t, fdst_t, fidx_src, fidx_dst, dst,
                               attn_t, zeros_flat)
    gat8 = _gat_agg_sc(fsrc_t, fidx_src, dst, ex, denom, zeros)
    out = (gat8.reshape(4, 2, NP, 128)[:, :, :N]
           .transpose(2, 0, 1, 3).reshape(N, HEADS, H))
    out = out + gat_bias.reshape(1, HEADS, H)
    h1 = jax.nn.elu(out).reshape(N, HEADS * H)
    skip = x @ skip_W.T + skip_b
    h1 = h1 + skip
    h1 = (h1 - h1.mean(0)) / jnp.sqrt(h1.var(0) + 1e-5) * bn1_g + bn1_b
    hr = h1 @ red_W.T + red_b
    gate = hr @ gate_W.T + gate_b
    gate = jax.nn.softmax(gate, axis=0)
    h_g = (gate * hr).sum(0, keepdims=True)
    z1 = jax.nn.relu(h_g @ cls_W1.T + cls_b1)
    return z1 @ cls_W2.T + cls_b2
